# speedup vs baseline: 20.4883x; 20.4883x over previous
import jax
import jax.numpy as jnp
from jax.experimental import pallas as pl
from jax.experimental.pallas import tpu as pltpu
from jax.experimental.pallas import tpu_sc as plsc

_WINDOW = 128


def _gather_sc(table, gidx, num_rows, dim):
    mesh = plsc.VectorSubcoreMesh(core_axis_name="core", subcore_axis_name="subcore")

    @pl.kernel(
        out_type=jax.ShapeDtypeStruct((num_rows, dim), jnp.float32),
        mesh=mesh,
    )
    def k(x_hbm, i_hbm, o_hbm):
        def body(i_vmem, o_vmem):
            pltpu.sync_copy(x_hbm.at[i_vmem.at[0]], o_vmem)

        pltpu.emit_pipeline(
            body,
            grid=(num_rows // _WINDOW,),
            in_specs=[pl.BlockSpec((1, _WINDOW), lambda i: (0, i))],
            out_specs=[pl.BlockSpec((_WINDOW, dim), lambda i: (i, 0))],
            core_axis_name=("core", "subcore"),
            dimension_semantics=(pltpu.PARALLEL,),
        )(i_hbm, o_hbm)

    return k(table, gidx)


def kernel(graph_lstm_output, slic_output):
    B, S, D = graph_lstm_output.shape
    _, H, W = slic_output.shape
    n = B * H * W
    ext = jnp.concatenate(
        [jnp.zeros((B, 1, D), graph_lstm_output.dtype), graph_lstm_output], axis=1
    ).reshape(B * (S + 1), D)
    offs = (jnp.arange(B, dtype=jnp.int32) * (S + 1))[:, None]
    gidx = (slic_output.reshape(B, H * W) + offs).reshape(1, n)
    out = _gather_sc(ext, gidx, n, D)
    return out.reshape(B, H, W, D)


# speedup vs baseline: 20.5371x; 1.0024x over previous
import jax
import jax.numpy as jnp
from jax.experimental import pallas as pl
from jax.experimental.pallas import tpu as pltpu
from jax.experimental.pallas import tpu_sc as plsc

_WINDOW = 384


def _gather_sc(table, gidx, num_rows, dim):
    mesh = plsc.VectorSubcoreMesh(core_axis_name="core", subcore_axis_name="subcore")

    @pl.kernel(
        out_type=jax.ShapeDtypeStruct((num_rows, dim), jnp.float32),
        mesh=mesh,
    )
    def k(x_hbm, i_hbm, o_hbm):
        def body(i_vmem, o_vmem):
            for j in range(_WINDOW // 128):
                pltpu.sync_copy(
                    x_hbm.at[i_vmem.at[0, pl.ds(j * 128, 128)]],
                    o_vmem.at[pl.ds(j * 128, 128)],
                )

        pltpu.emit_pipeline(
            body,
            grid=(num_rows // _WINDOW,),
            in_specs=[pl.BlockSpec((1, _WINDOW), lambda i: (0, i))],
            out_specs=[pl.BlockSpec((_WINDOW, dim), lambda i: (i, 0))],
            core_axis_name=("core", "subcore"),
            dimension_semantics=(pltpu.PARALLEL,),
        )(i_hbm, o_hbm)

    return k(table, gidx)


def kernel(graph_lstm_output, slic_output):
    B, S, D = graph_lstm_output.shape
    _, H, W = slic_output.shape
    n = B * H * W
    ext = jnp.concatenate(
        [jnp.zeros((B, 1, D), graph_lstm_output.dtype), graph_lstm_output], axis=1
    ).reshape(B * (S + 1), D)
    offs = (jnp.arange(B, dtype=jnp.int32) * (S + 1))[:, None]
    gidx = (slic_output.reshape(B, H * W) + offs).reshape(1, n)
    out = _gather_sc(ext, gidx, n, D)
    return out.reshape(B, H, W, D)


# speedup vs baseline: 21.6904x; 1.0562x over previous
import jax
import jax.numpy as jnp
from jax import lax
from jax.experimental import pallas as pl
from jax.experimental.pallas import tpu as pltpu
from jax.experimental.pallas import tpu_sc as plsc

_NC, _NS = 2, 16
_NW = _NC * _NS
_G = 128
_NB = 4


def _gather_sc(table, gidx, num_rows, dim):
    per_w = num_rows // _NW
    K = per_w // _G
    G_GROUPS = K // _NB
    mesh = plsc.VectorSubcoreMesh(core_axis_name="core", subcore_axis_name="subcore")

    @pl.kernel(
        out_type=jax.ShapeDtypeStruct((num_rows, dim), jnp.float32),
        mesh=mesh,
        scratch_types=[
            pltpu.VMEM((K, _G), jnp.int32),
            pltpu.VMEM((_NB, _G, dim), jnp.float32),
            pltpu.SemaphoreType.DMA,
            pltpu.SemaphoreType.DMA,
        ],
    )
    def k(x_hbm, i_hbm, o_hbm, idx_v, rows_v, gsem, wsem):
        wid = lax.axis_index("subcore") * _NC + lax.axis_index("core")
        base = wid * per_w

        def gather(c, j):
            return pltpu.make_async_copy(x_hbm.at[idx_v.at[c]], rows_v.at[j], gsem)

        def write(c, j):
            return pltpu.make_async_copy(
                rows_v.at[j], o_hbm.at[pl.ds(base + c * _G, _G)], wsem
            )

        pltpu.sync_copy(i_hbm.at[wid], idx_v)
        for j in range(_NB):
            gather(j, j).start()

        @pl.loop(0, G_GROUPS - 1)
        def _(g):
            c0 = g * _NB
            for j in range(_NB):
                gather(c0 + j, j).wait()
                write(c0 + j, j).start()
            for j in range(_NB):
                write(c0 + j, j).wait()
                gather(c0 + _NB + j, j).start()

        c0 = (G_GROUPS - 1) * _NB
        for j in range(_NB):
            gather(c0 + j, j).wait()
            write(c0 + j, j).start()
        for j in range(_NB):
            write(c0 + j, j).wait()

    return k(table, gidx)


def kernel(graph_lstm_output, slic_output):
    B, S, D = graph_lstm_output.shape
    _, H, W = slic_output.shape
    n = B * H * W
    ext = jnp.concatenate(
        [jnp.zeros((B, 1, D), graph_lstm_output.dtype), graph_lstm_output], axis=1
    ).reshape(B * (S + 1), D)
    offs = (jnp.arange(B, dtype=jnp.int32) * (S + 1))[:, None]
    gidx = (slic_output.reshape(B, H * W) + offs).reshape(_NW, n // (_NW * _G), _G)
    out = _gather_sc(ext, gidx, n, D)
    return out.reshape(B, H, W, D)


# speedup vs baseline: 56.2124x; 2.5916x over previous
import jax
import jax.numpy as jnp
from jax import lax
from jax.experimental import pallas as pl
from jax.experimental.pallas import tpu as pltpu
from jax.experimental.pallas import tpu_sc as plsc

_NC, _NS = 2, 16
_NW = _NC * _NS
_G = 128
_NB = 4


def _gather_sc(table, gidx, num_rows, dim):
    per_w = num_rows // _NW
    K = per_w // _G
    G_GROUPS = K // _NB
    mesh = plsc.VectorSubcoreMesh(core_axis_name="core", subcore_axis_name="subcore")

    V = table.shape[0]

    @pl.kernel(
        out_type=jax.ShapeDtypeStruct((num_rows, dim), jnp.float32),
        mesh=mesh,
        scratch_types=[
            pltpu.VMEM((K, _G), jnp.int32),
            pltpu.VMEM((_NB, _G, dim), jnp.float32),
            pltpu.VMEM_SHARED((V, dim), jnp.float32),
            pltpu.SemaphoreType.DMA,
            pltpu.SemaphoreType.DMA,
            pltpu.SemaphoreType.DMA,
        ],
    )
    def k(x_hbm, i_hbm, o_hbm, idx_v, rows_v, tab_s, gsem, wsem, tsem):
        wid = lax.axis_index("subcore") * _NC + lax.axis_index("core")
        base = wid * per_w

        @pl.when(lax.axis_index("subcore") == 0)
        def _():
            pltpu.async_copy(x_hbm, tab_s, tsem).wait()

        plsc.subcore_barrier()

        def gather(c, j):
            return pltpu.make_async_copy(tab_s.at[idx_v.at[c]], rows_v.at[j], gsem)

        def write(c, j):
            return pltpu.make_async_copy(
                rows_v.at[j], o_hbm.at[pl.ds(base + c * _G, _G)], wsem
            )

        pltpu.sync_copy(i_hbm.at[wid], idx_v)
        for j in range(_NB):
            gather(j, j).start()

        @pl.loop(0, G_GROUPS - 1)
        def _(g):
            c0 = g * _NB
            for j in range(_NB):
                gather(c0 + j, j).wait()
                write(c0 + j, j).start()
            for j in range(_NB):
                write(c0 + j, j).wait()
                gather(c0 + _NB + j, j).start()

        c0 = (G_GROUPS - 1) * _NB
        for j in range(_NB):
            gather(c0 + j, j).wait()
            write(c0 + j, j).start()
        for j in range(_NB):
            write(c0 + j, j).wait()

    return k(table, gidx)


def kernel(graph_lstm_output, slic_output):
    B, S, D = graph_lstm_output.shape
    _, H, W = slic_output.shape
    n = B * H * W
    ext = jnp.concatenate(
        [jnp.zeros((B, 1, D), graph_lstm_output.dtype), graph_lstm_output], axis=1
    ).reshape(B * (S + 1), D)
    offs = (jnp.arange(B, dtype=jnp.int32) * (S + 1))[:, None]
    gidx = (slic_output.reshape(B, H * W) + offs).reshape(_NW, n // (_NW * _G), _G)
    out = _gather_sc(ext, gidx, n, D)
    return out.reshape(B, H, W, D)


# speedup vs baseline: 57.6618x; 1.0258x over previous
import jax
import jax.numpy as jnp
from jax import lax
from jax.experimental import pallas as pl
from jax.experimental.pallas import tpu as pltpu
from jax.experimental.pallas import tpu_sc as plsc

_NC, _NS = 2, 16
_NW = _NC * _NS
_G = 128
_NB = 4


def _gather_sc(table, slic, B, num_rows, dim):
    S = table.shape[1]
    per_w = num_rows // _NW
    K = per_w // _G
    G_GROUPS = K // _NB
    w_per_b = _NW // B
    mesh = plsc.VectorSubcoreMesh(core_axis_name="core", subcore_axis_name="subcore")

    @pl.kernel(
        out_type=jax.ShapeDtypeStruct((num_rows, dim), jnp.float32),
        mesh=mesh,
        scratch_types=[
            pltpu.VMEM((K, _G), jnp.int32),
            pltpu.VMEM((_NB, _G, dim), jnp.float32),
            pltpu.VMEM((1, dim), jnp.float32),
            pltpu.VMEM_SHARED((B, S + 1, dim), jnp.float32),
            pltpu.SemaphoreType.DMA,
            pltpu.SemaphoreType.DMA,
            pltpu.SemaphoreType.DMA,
        ],
    )
    def k(x_hbm, i_hbm, o_hbm, idx_v, rows_v, zrow_v, tab_s, gsem, wsem, tsem):
        cid = lax.axis_index("core")
        sid = lax.axis_index("subcore")
        wid = sid * _NC + cid
        base = wid * per_w
        batch = wid // w_per_b

        pltpu.async_copy(i_hbm.at[wid], idx_v, gsem)

        @pl.when(sid < B)
        def _():
            for c in range(dim // 16):
                zrow_v[0, pl.ds(c * 16, 16)] = jnp.zeros((16,), jnp.float32)
            pltpu.async_copy(x_hbm.at[sid], tab_s.at[sid, pl.ds(1, S)], tsem)
            pltpu.async_copy(zrow_v, tab_s.at[sid, pl.ds(0, 1)], tsem)
            pltpu.make_async_copy(x_hbm.at[sid], tab_s.at[sid, pl.ds(1, S)], tsem).wait()
            pltpu.make_async_copy(zrow_v, tab_s.at[sid, pl.ds(0, 1)], tsem).wait()

        pltpu.make_async_copy(i_hbm.at[wid], idx_v, gsem).wait()
        plsc.subcore_barrier()

        tab = tab_s.at[batch]

        def gather(c, j):
            return pltpu.make_async_copy(tab.at[idx_v.at[c]], rows_v.at[j], gsem)

        def write(c, j):
            return pltpu.make_async_copy(
                rows_v.at[j], o_hbm.at[pl.ds(base + c * _G, _G)], wsem
            )

        for j in range(_NB):
            gather(j, j).start()

        @pl.loop(0, G_GROUPS - 1)
        def _(g):
            c0 = g * _NB
            for j in range(_NB):
                gather(c0 + j, j).wait()
                write(c0 + j, j).start()
            for j in range(_NB):
                write(c0 + j, j).wait()
                gather(c0 + _NB + j, j).start()

        c0 = (G_GROUPS - 1) * _NB
        for j in range(_NB):
            gather(c0 + j, j).wait()
            write(c0 + j, j).start()
        for j in range(_NB):
            write(c0 + j, j).wait()

    return k(table, slic)


def kernel(graph_lstm_output, slic_output):
    B, S, D = graph_lstm_output.shape
    _, H, W = slic_output.shape
    n = B * H * W
    slic = slic_output.reshape(_NW, n // (_NW * _G), _G)
    out = _gather_sc(graph_lstm_output, slic, B, n, D)
    return out.reshape(B, H, W, D)
